# baseline (speedup 1.0000x reference)
import functools

import jax
import jax.numpy as jnp
import numpy as np
from jax.experimental import pallas as pl
from jax.experimental.pallas import tpu as pltpu

TQ = 128
HALO = 448
KW = TQ + HALO
BLK = 64
NEG = -1e30
NSPECIAL = -(-HALO // TQ)


def _bias_table():
    planes = []
    for t in range(NSPECIAL + 1):
        q0 = t * TQ
        start = max(q0 - HALO, 0)
        r = q0 + np.arange(TQ)[:, None]
        c = start + np.arange(KW)[None, :]
        allowed = (c <= r) & (c >= (r // BLK) * BLK - HALO)
        planes.append(np.where(allowed, 0.0, NEG).astype(np.float32))
    return np.stack(planes)


def _attn_tile(q_ref, k_ref, v_ref, b_ref, o_ref, *, scale):
    i = pl.program_id(1)
    q0 = i * TQ
    start = jnp.maximum(q0 - HALO, 0)

    q = (q_ref[0] * scale).astype(jnp.bfloat16)
    ks = k_ref[0, pl.ds(start, KW), :].astype(jnp.bfloat16)
    vs = v_ref[0, pl.ds(start, KW), :].astype(jnp.bfloat16)
    bias = b_ref[jnp.minimum(i, NSPECIAL)]

    scores = jax.lax.dot_general(
        q, ks, (((1,), (1,)), ((), ())),
        preferred_element_type=jnp.float32)

    e = jnp.exp(scores + bias)
    denom = jnp.sum(e, axis=-1, keepdims=True)

    acc = jax.lax.dot_general(
        e.astype(jnp.bfloat16), vs, (((1,), (0,)), ((), ())),
        preferred_element_type=jnp.float32)
    o_ref[0] = acc / denom


def kernel(q, k, v):
    B, H, S, D = q.shape
    scale = float(D) ** -0.5
    qf = q.reshape(B * H, S, D)
    kf = k.reshape(B * H, S, D)
    vf = v.reshape(B * H, S, D)
    bias = jnp.asarray(_bias_table())

    grid = (B * H, S // TQ)
    out = pl.pallas_call(
        functools.partial(_attn_tile, scale=scale),
        grid=grid,
        in_specs=[
            pl.BlockSpec((1, TQ, D), lambda h, i: (h, i, 0)),
            pl.BlockSpec((1, S, D), lambda h, i: (h, 0, 0)),
            pl.BlockSpec((1, S, D), lambda h, i: (h, 0, 0)),
            pl.BlockSpec((NSPECIAL + 1, TQ, KW), lambda h, i: (0, 0, 0)),
        ],
        out_specs=pl.BlockSpec((1, TQ, D), lambda h, i: (h, i, 0)),
        out_shape=jax.ShapeDtypeStruct((B * H, S, D), jnp.float32),
        compiler_params=pltpu.CompilerParams(
            dimension_semantics=("arbitrary", "arbitrary"),
        ),
    )(qf, kf, vf, bias)
    return out.reshape(B, H, S, D)


# speedup vs baseline: 4.4927x; 4.4927x over previous
import functools

import jax
import jax.numpy as jnp
from jax.experimental import pallas as pl
from jax.experimental.pallas import tpu as pltpu

TQ = 256
HALO = 448
KW = TQ + HALO
BLK = 64
NEG = -1e30
NSPECIAL = -(-HALO // TQ)


def _attn_head(q_ref, k_ref, v_ref, o_ref, b_ref, *, scale, seq_len):
    h = pl.program_id(0)

    @pl.when(h == 0)
    def _init_bias():
        for t in range(NSPECIAL + 1):
            q0 = t * TQ
            start = max(q0 - HALO, 0)
            r = jax.lax.broadcasted_iota(jnp.int32, (TQ, KW), 0) + q0
            c = jax.lax.broadcasted_iota(jnp.int32, (TQ, KW), 1) + start
            allowed = (c <= r) & (c >= (r // BLK) * BLK - HALO)
            b_ref[t] = jnp.where(allowed, 0.0, NEG)

    kbf = k_ref[0].astype(jnp.bfloat16)
    vbf = v_ref[0].astype(jnp.bfloat16)

    for i in range(seq_len // TQ):
        q0 = i * TQ
        start = max(q0 - HALO, 0)
        q = (q_ref[0, q0:q0 + TQ, :] * scale).astype(jnp.bfloat16)
        ks = kbf[start:start + KW]
        vs = vbf[start:start + KW]
        bias = b_ref[min(i, NSPECIAL)]

        scores = jax.lax.dot_general(
            q, ks, (((1,), (1,)), ((), ())),
            preferred_element_type=jnp.float32)

        e = jnp.exp(scores + bias)
        denom = jnp.sum(e, axis=-1, keepdims=True)

        acc = jax.lax.dot_general(
            e.astype(jnp.bfloat16), vs, (((1,), (0,)), ((), ())),
            preferred_element_type=jnp.float32)
        o_ref[0, q0:q0 + TQ, :] = acc / denom


def kernel(q, k, v):
    B, H, S, D = q.shape
    scale = float(D) ** -0.5
    qf = q.reshape(B * H, S, D)
    kf = k.reshape(B * H, S, D)
    vf = v.reshape(B * H, S, D)

    out = pl.pallas_call(
        functools.partial(_attn_head, scale=scale, seq_len=S),
        grid=(B * H,),
        in_specs=[
            pl.BlockSpec((1, S, D), lambda h: (h, 0, 0)),
            pl.BlockSpec((1, S, D), lambda h: (h, 0, 0)),
            pl.BlockSpec((1, S, D), lambda h: (h, 0, 0)),
        ],
        out_specs=pl.BlockSpec((1, S, D), lambda h: (h, 0, 0)),
        out_shape=jax.ShapeDtypeStruct((B * H, S, D), jnp.float32),
        scratch_shapes=[pltpu.VMEM((NSPECIAL + 1, TQ, KW), jnp.float32)],
        compiler_params=pltpu.CompilerParams(
            dimension_semantics=("arbitrary",),
        ),
    )(qf, kf, vf)
    return out.reshape(B, H, S, D)
